# baseline (speedup 1.0000x reference)
import functools

import jax
import jax.numpy as jnp
from jax import lax
from jax.experimental import pallas as pl
from jax.experimental.pallas import tpu as pltpu
from jax.experimental.pallas import tpu_sc as plsc

_INCREASE_DELTA = 0.05
_ABS_THRESHOLD = 0.8

_B = 16
_S = 4096
_L = 16
_NCHUNK = _S // _L
_PAD = _L
_NEG = -3e38
_POS = 3e38


def _seg_body(ent_hbm, res_hbm, row_v, res_v):
    wid = lax.axis_index("s")

    rb = wid * _S
    row_v[pl.ds(0, _L)] = jnp.full((_L,), _NEG, jnp.float32)
    pltpu.sync_copy(ent_hbm.at[pl.ds(rb, _S)], row_v.at[pl.ds(_PAD, _S)])

    lane = lax.iota(jnp.int32, _L)

    def chunk(i, carry):
        carry_sum, carry_max = carry
        base = _PAD + i * _L
        prev = row_v[pl.ds(base - 1, _L)]
        e = row_v[pl.ds(base, _L)]
        inc = (e > prev + _INCREASE_DELTA) | (e > _ABS_THRESHOLD)
        inc_i = inc.astype(jnp.int32)
        off = i * _L
        res_v[pl.ds(off, _L)] = plsc.cumsum(inc_i) + carry_sum
        fp = jnp.where(inc, off + lane, 0)
        res_v[pl.ds(_S + off, _L)] = jnp.maximum(plsc.cummax(fp), carry_max)
        cnt = plsc.all_reduce_population_count(inc)
        ffs = plsc.all_reduce_ffs(lax.rev(inc_i, (0,)) != 0)
        new_max = jnp.where(cnt > 0, (off + 15) - ffs, carry_max)
        return carry_sum + cnt, new_max

    lax.fori_loop(
        0, _NCHUNK, chunk,
        (jnp.full((_L,), -1, jnp.int32), jnp.zeros((_L,), jnp.int32)),
    )

    pltpu.sync_copy(res_v, res_hbm.at[pl.ds(wid * (2 * _S), 2 * _S)])


def _stage_in_body(ent_ref, pem_ref, flat_ref):
    e = ent_ref[...]
    nxt = jnp.concatenate(
        [e[:, 1:], jnp.full((8, 1), _POS, jnp.float32)], axis=1
    )
    pem_ref[...] = (nxt > e + _INCREASE_DELTA) | (nxt > _ABS_THRESHOLD)
    flat_ref[...] = e.reshape(8 * _S)


def _stage_out_body(res_ref, seg_ref, fb_ref):
    x = res_ref[...].reshape(8, 2, _S)
    seg_ref[...] = x[:, 0, :]
    fb_ref[...] = x[:, 1, :]


@jax.jit
def _segmenter(entropy_bits):
    pem, ent_flat = pl.pallas_call(
        _stage_in_body,
        grid=(2,),
        in_specs=[pl.BlockSpec((8, _S), lambda w: (w, 0))],
        out_specs=(
            pl.BlockSpec((8, _S), lambda w: (w, 0)),
            pl.BlockSpec((8 * _S,), lambda w: (w,)),
        ),
        out_shape=(
            jax.ShapeDtypeStruct((_B, _S), jnp.bool_),
            jax.ShapeDtypeStruct((_B * _S,), jnp.float32),
        ),
    )(entropy_bits)

    mesh = plsc.VectorSubcoreMesh(
        core_axis_name="c", subcore_axis_name="s", num_cores=1, num_subcores=16
    )
    run = functools.partial(
        pl.kernel,
        out_type=jax.ShapeDtypeStruct((_B * 2 * _S,), jnp.int32),
        mesh=mesh,
        compiler_params=pltpu.CompilerParams(
            needs_layout_passes=False, skip_device_barrier=True
        ),
        scratch_types=[
            pltpu.VMEM((_PAD + _S,), jnp.float32),
            pltpu.VMEM((2 * _S,), jnp.int32),
        ],
    )(_seg_body)
    res = run(ent_flat)

    seg, fb = pl.pallas_call(
        _stage_out_body,
        grid=(2,),
        in_specs=[pl.BlockSpec((8 * 2 * _S,), lambda w: (w,))],
        out_specs=(
            pl.BlockSpec((8, _S), lambda w: (w, 0)),
            pl.BlockSpec((8, _S), lambda w: (w, 0)),
        ),
        out_shape=(
            jax.ShapeDtypeStruct((_B, _S), jnp.int32),
            jax.ShapeDtypeStruct((_B, _S), jnp.int32),
        ),
    )(res)
    return seg, pem, fb


def kernel(entropy_bits):
    return _segmenter(entropy_bits)
